# Initial kernel scaffold; baseline (speedup 1.0000x reference)
import jax
import jax.numpy as jnp
from jax.experimental import pallas as pl


def kernel(node_id, node_sno, edge_index, fixed_embeddings, W_msg, W_ih, W_hh, W_out):
    raise NotImplementedError("write your pallas kernel here")


# speedup vs baseline: 3.3192x; 3.3192x over previous
import functools

import jax
import jax.numpy as jnp
from jax import lax
from jax.experimental import pallas as pl
from jax.experimental.pallas import tpu as pltpu
from jax.experimental.pallas import tpu_sc as plsc

H = 128
B = 100
BOARD = 99
N = B * (BOARD + 1)
E = 320000
T = 3

NC = 2
NS = 16
NW = NC * NS

CHUNK = 128
ECH_PER_TILE = 80
NPAD = 10240
ROWS_PER_TILE = NPAD // NS
LCHUNKS = (N + CHUNK - 1) // CHUNK
LCH_ROUND = -(-LCHUNKS // NW)
LPAD = LCHUNKS * CHUNK


@functools.cache
def _build_sc_lookup():
    mesh = plsc.VectorSubcoreMesh(core_axis_name="c", subcore_axis_name="s",
                                  num_cores=NC)

    @functools.partial(
        pl.kernel,
        out_type=jax.ShapeDtypeStruct((LPAD, H), jnp.float32),
        mesh=mesh,
        scratch_types=[
            pltpu.VMEM((CHUNK,), jnp.int32),
            pltpu.VMEM((CHUNK, H), jnp.float32),
            pltpu.SemaphoreType.DMA,
        ],
    )
    def sc_lookup(emb_hbm, idx_hbm, out_hbm, idx_v, rows_v, sem):
        cid = lax.axis_index("c")
        sid = lax.axis_index("s")
        wid = sid * NC + cid
        for r in range(LCH_ROUND):
            chunk = wid + NW * r

            @pl.when(chunk < LCHUNKS)
            def _():
                pltpu.sync_copy(idx_hbm.at[chunk], idx_v)
                pltpu.async_copy(emb_hbm.at[idx_v], rows_v, sem).wait()
                pltpu.sync_copy(rows_v, out_hbm.at[pl.ds(chunk * CHUNK, CHUNK)])

    return sc_lookup


def _sc_lookup(emb, idx_pad):
    return _build_sc_lookup()(emb, idx_pad)


@functools.cache
def _build_sc_segsum():
    mesh = plsc.VectorSubcoreMesh(core_axis_name="c", subcore_axis_name="s",
                                  num_cores=NC)

    @functools.partial(
        pl.kernel,
        out_type=jax.ShapeDtypeStruct((NC, NPAD, H), jnp.float32),
        mesh=mesh,
        scratch_types=[
            pltpu.VMEM((ECH_PER_TILE, CHUNK), jnp.int32),
            pltpu.VMEM((ECH_PER_TILE, CHUNK), jnp.int32),
            pltpu.VMEM((CHUNK, H), jnp.float32),
            pltpu.VMEM_SHARED((NPAD, H), jnp.float32),
            pltpu.SemaphoreType.DMA,
        ],
    )
    def sc_segsum(h_hbm, src_hbm, dst_hbm, zeros_hbm, out_hbm,
                  src_v, dst_v, rows_v, accum, sem):
        cid = lax.axis_index("c")
        sid = lax.axis_index("s")
        wid = sid * NC + cid

        pltpu.sync_copy(src_hbm.at[wid], src_v)
        pltpu.sync_copy(dst_hbm.at[wid], dst_v)

        pltpu.sync_copy(zeros_hbm, rows_v)
        for k in range(ROWS_PER_TILE // CHUNK):
            pltpu.sync_copy(rows_v,
                            accum.at[pl.ds(sid * ROWS_PER_TILE + k * CHUNK, CHUNK)])
        plsc.subcore_barrier()

        def body(j, carry):
            pltpu.async_copy(h_hbm.at[src_v.at[j]], rows_v, sem).wait()
            pltpu.sync_copy(rows_v, accum.at[dst_v.at[j]], add=True)
            return carry

        lax.fori_loop(0, ECH_PER_TILE, body, 0)
        plsc.subcore_barrier()

        base = sid * ROWS_PER_TILE
        for k in range(ROWS_PER_TILE // CHUNK):
            pltpu.sync_copy(accum.at[pl.ds(base + k * CHUNK, CHUNK)], rows_v)
            pltpu.sync_copy(rows_v, out_hbm.at[cid, pl.ds(base + k * CHUNK, CHUNK)])

    return sc_segsum


def _sc_segsum(h, src_pad, dst_pad, zeros_blk):
    return _build_sc_segsum()(h, src_pad, dst_pad, zeros_blk)


BLK = 2000
NB_BLK = BLK // (BOARD + 1)


def _lstm_block(x_ref, s0_ref, s1_ref, h_ref, c_ref, wmsgT_ref, wihT_ref,
                whhT_ref, h_out, c_out):
    s = s0_ref[0] + s1_ref[0]
    m = jnp.dot(s, wmsgT_ref[...], preferred_element_type=jnp.float32)
    wihT = wihT_ref[...]
    gates = (jnp.dot(x_ref[...], wihT[:H], preferred_element_type=jnp.float32)
             + jnp.dot(m, wihT[H:], preferred_element_type=jnp.float32)
             + jnp.dot(h_ref[...], whhT_ref[...], preferred_element_type=jnp.float32))
    gi = gates[:, :H]
    gf = gates[:, H:2 * H]
    gg = gates[:, 2 * H:3 * H]
    go = gates[:, 3 * H:]
    c_new = jax.nn.sigmoid(gf) * c_ref[...] + jax.nn.sigmoid(gi) * jnp.tanh(gg)
    h_out[...] = jax.nn.sigmoid(go) * jnp.tanh(c_new)
    c_out[...] = c_new


def _tc_lstm(x, sc_out, h, c, wmsgT, wihT, whhT):
    row_spec = pl.BlockSpec((BLK, H), lambda i: (i, 0))
    full2 = lambda shape: pl.BlockSpec(shape, lambda i: (0, 0))
    return pl.pallas_call(
        _lstm_block,
        grid=(N // BLK,),
        in_specs=[
            row_spec,
            pl.BlockSpec((1, BLK, H), lambda i: (0, i, 0)),
            pl.BlockSpec((1, BLK, H), lambda i: (1, i, 0)),
            row_spec,
            row_spec,
            full2((H, H)),
            full2((2 * H, 4 * H)),
            full2((H, 4 * H)),
        ],
        out_specs=[row_spec, row_spec],
        out_shape=[jax.ShapeDtypeStruct((N, H), jnp.float32),
                   jax.ShapeDtypeStruct((N, H), jnp.float32)],
    )(x, sc_out, sc_out, h, c, wmsgT, wihT, whhT)


def _gol_sum(v, sel):
    n2 = jnp.sum(v * v, axis=1, keepdims=True)
    inv = 1.0 / (jnp.sqrt(n2) + 1e-8)
    vn = v * inv
    bs = jnp.dot(sel, vn, preferred_element_type=jnp.float32)
    tr = jnp.sum(sel * jnp.transpose(n2 * inv * inv))
    return (jnp.sum(bs * bs) - tr) / (BOARD * (BOARD - 1))


def _final_block(h1_ref, h2_ref, h3_ref, emb_ref, woutT_ref,
                 gol_out, in_out, out_out):
    i = pl.program_id(0)
    h3v = h3_ref[...]
    y = jnp.dot(h3v, woutT_ref[...], preferred_element_type=jnp.float32)
    rows = lax.broadcasted_iota(jnp.int32, (BLK, 1), 0)
    mask0 = (rows % (BOARD + 1)) == 0
    embv = emb_ref[...]
    in_out[...] = jnp.where(mask0, embv, h3v)
    out_out[...] = jnp.where(mask0, embv, y)

    colid = lax.broadcasted_iota(jnp.int32, (NB_BLK, BLK), 1)
    rowid = lax.broadcasted_iota(jnp.int32, (NB_BLK, BLK), 0)
    sel = jnp.where((colid // (BOARD + 1) == rowid)
                    & (colid % (BOARD + 1) != 0), 1.0, 0.0)

    part = ((_gol_sum(h1_ref[...], sel) + _gol_sum(h2_ref[...], sel)
             + _gol_sum(h3v, sel)) / (T * B)
            + _gol_sum(y, sel) / B)

    @pl.when(i == 0)
    def _():
        gol_out[...] = jnp.zeros_like(gol_out)

    gol_out[...] += part


def _tc_final(h1, h2, h3, emb_n, woutT):
    row_spec = pl.BlockSpec((BLK, H), lambda i: (i, 0))
    return pl.pallas_call(
        _final_block,
        grid=(N // BLK,),
        in_specs=[row_spec, row_spec, row_spec, row_spec,
                  pl.BlockSpec((H, H), lambda i: (0, 0))],
        out_specs=[pl.BlockSpec((1, 1), lambda i: (0, 0)), row_spec, row_spec],
        out_shape=[jax.ShapeDtypeStruct((1, 1), jnp.float32),
                   jax.ShapeDtypeStruct((N, H), jnp.float32),
                   jax.ShapeDtypeStruct((N, H), jnp.float32)],
    )(h1, h2, h3, emb_n, woutT)


def kernel(node_id, node_sno, edge_index, fixed_embeddings, W_msg, W_ih, W_hh, W_out):
    emb_n = fixed_embeddings[:N]

    lookup_at = (node_id + (BOARD + 1) * node_sno).astype(jnp.int32)
    idx_pad = jnp.pad(lookup_at, (0, LPAD - N)).reshape(LCHUNKS, CHUNK)
    x_pad = _sc_lookup(fixed_embeddings, idx_pad)
    x = x_pad[:N]

    src = edge_index[0].astype(jnp.int32).reshape(NW, E // NW)
    dst = edge_index[1].astype(jnp.int32).reshape(NW, E // NW)
    pad_e = ECH_PER_TILE * CHUNK - E // NW
    src_pad = jnp.pad(src, ((0, 0), (0, pad_e))).reshape(NW, ECH_PER_TILE, CHUNK)
    dst_pad = jnp.pad(dst, ((0, 0), (0, pad_e)),
                      constant_values=N).reshape(NW, ECH_PER_TILE, CHUNK)
    zeros_blk = jnp.zeros((CHUNK, H), jnp.float32)

    wmsgT = W_msg.T
    wihT = W_ih.T
    whhT = W_hh.T

    h = x
    c = x
    steps = []
    for _ in range(T):
        sc_out = _sc_segsum(h, src_pad, dst_pad, zeros_blk)
        h, c = _tc_lstm(x, sc_out, h, c, wmsgT, wihT, whhT)
        steps.append(h)

    gol_arr, in_final, out_final = _tc_final(steps[0], steps[1], steps[2],
                                             emb_n, W_out.T)
    gol = gol_arr[0, 0]
    step_input = jnp.stack(steps, axis=0)
    return (gol, emb_n, in_final, out_final, step_input)


# speedup vs baseline: 3.6051x; 1.0861x over previous
import functools

import jax
import jax.numpy as jnp
from jax import lax
from jax.experimental import pallas as pl
from jax.experimental.pallas import tpu as pltpu
from jax.experimental.pallas import tpu_sc as plsc

H = 128
B = 100
BOARD = 99
N = B * (BOARD + 1)
E = 320000
T = 3

NC = 2
NS = 16
NW = NC * NS

CHUNK = 128
ECH_PER_TILE = 80
NPAD = 10240
ROWS_PER_TILE = NPAD // NS
LCHUNKS = (N + CHUNK - 1) // CHUNK
LCH_ROUND = -(-LCHUNKS // NW)
LPAD = LCHUNKS * CHUNK


@functools.cache
def _build_sc_lookup():
    mesh = plsc.VectorSubcoreMesh(core_axis_name="c", subcore_axis_name="s",
                                  num_cores=NC)

    @functools.partial(
        pl.kernel,
        out_type=jax.ShapeDtypeStruct((LPAD, H), jnp.float32),
        mesh=mesh,
        scratch_types=[
            pltpu.VMEM((CHUNK,), jnp.int32),
            pltpu.VMEM((CHUNK, H), jnp.float32),
            pltpu.SemaphoreType.DMA,
        ],
    )
    def sc_lookup(emb_hbm, idx_hbm, out_hbm, idx_v, rows_v, sem):
        cid = lax.axis_index("c")
        sid = lax.axis_index("s")
        wid = sid * NC + cid
        for r in range(LCH_ROUND):
            chunk = wid + NW * r

            @pl.when(chunk < LCHUNKS)
            def _():
                pltpu.sync_copy(idx_hbm.at[chunk], idx_v)
                pltpu.async_copy(emb_hbm.at[idx_v], rows_v, sem).wait()
                pltpu.sync_copy(rows_v, out_hbm.at[pl.ds(chunk * CHUNK, CHUNK)])

    return sc_lookup


def _sc_lookup(emb, idx_pad):
    return _build_sc_lookup()(emb, idx_pad)


NBLK = 5
BCH = ECH_PER_TILE // NBLK


@functools.cache
def _build_sc_segsum():
    mesh = plsc.VectorSubcoreMesh(core_axis_name="c", subcore_axis_name="s",
                                  num_cores=NC)

    @functools.partial(
        pl.kernel,
        out_type=jax.ShapeDtypeStruct((NC, NPAD, H), jnp.float32),
        mesh=mesh,
        scratch_types=[
            pltpu.VMEM((2, BCH, CHUNK), jnp.int32),
            pltpu.VMEM((2, BCH, CHUNK), jnp.int32),
            pltpu.VMEM((CHUNK, H), jnp.float32),
            pltpu.VMEM((CHUNK, H), jnp.float32),
            pltpu.VMEM_SHARED((NPAD, H), jnp.float32),
            pltpu.SemaphoreType.DMA,
            pltpu.SemaphoreType.DMA,
            pltpu.SemaphoreType.DMA,
            pltpu.SemaphoreType.DMA,
            pltpu.SemaphoreType.DMA,
        ],
    )
    def sc_segsum(h_hbm, src_hbm, dst_hbm, zeros_hbm, out_hbm,
                  src_v, dst_v, rows0, rows1, accum,
                  isem0, isem1, gsem0, gsem1, zsem):
        cid = lax.axis_index("c")
        sid = lax.axis_index("s")
        wid = sid * NC + cid
        rows = (rows0, rows1)
        gsem = (gsem0, gsem1)
        isem = (isem0, isem1)

        pltpu.async_copy(src_hbm.at[wid, pl.ds(0, BCH)], src_v.at[0], isem0)
        pltpu.async_copy(dst_hbm.at[wid, pl.ds(0, BCH)], dst_v.at[0], isem0)

        pltpu.sync_copy(zeros_hbm, rows0)
        for k in range(ROWS_PER_TILE // CHUNK):
            pltpu.async_copy(
                rows0, accum.at[pl.ds(sid * ROWS_PER_TILE + k * CHUNK, CHUNK)],
                zsem)
        for k in range(ROWS_PER_TILE // CHUNK):
            pltpu.make_async_copy(
                rows0, accum.at[pl.ds(sid * ROWS_PER_TILE + k * CHUNK, CHUNK)],
                zsem).wait()
        plsc.subcore_barrier()

        for b in range(NBLK):
            pb = b & 1
            if b + 1 < NBLK:
                pltpu.async_copy(src_hbm.at[wid, pl.ds((b + 1) * BCH, BCH)],
                                 src_v.at[1 - pb], isem[1 - pb])
                pltpu.async_copy(dst_hbm.at[wid, pl.ds((b + 1) * BCH, BCH)],
                                 dst_v.at[1 - pb], isem[1 - pb])
            pltpu.make_async_copy(src_hbm.at[wid, pl.ds(b * BCH, BCH)],
                                  src_v.at[pb], isem[pb]).wait()
            pltpu.make_async_copy(dst_hbm.at[wid, pl.ds(b * BCH, BCH)],
                                  dst_v.at[pb], isem[pb]).wait()

            pltpu.async_copy(h_hbm.at[src_v.at[pb, 0]], rows0, gsem0)

            def body(i, carry, pb=pb):
                for u in (0, 1):
                    j = 2 * i + u
                    pltpu.make_async_copy(h_hbm.at[src_v.at[pb, j]],
                                          rows[u], gsem[u]).wait()

                    @pl.when(j + 1 < BCH)
                    def _():
                        pltpu.async_copy(h_hbm.at[src_v.at[pb, j + 1]],
                                         rows[1 - u], gsem[1 - u])

                    pltpu.sync_copy(rows[u], accum.at[dst_v.at[pb, j]],
                                    add=True)
                return carry

            lax.fori_loop(0, BCH // 2, body, 0)
        plsc.subcore_barrier()

        base = sid * ROWS_PER_TILE
        for k in range(ROWS_PER_TILE // CHUNK):
            pltpu.sync_copy(accum.at[pl.ds(base + k * CHUNK, CHUNK)],
                            rows[k & 1])
            pltpu.sync_copy(rows[k & 1],
                            out_hbm.at[cid, pl.ds(base + k * CHUNK, CHUNK)])

    return sc_segsum


def _sc_segsum(h, src_pad, dst_pad, zeros_blk):
    return _build_sc_segsum()(h, src_pad, dst_pad, zeros_blk)


BLK = 2000
NB_BLK = BLK // (BOARD + 1)


def _lstm_block(x_ref, s0_ref, s1_ref, h_ref, c_ref, wmsgT_ref, wihT_ref,
                whhT_ref, h_out, c_out):
    s = s0_ref[0] + s1_ref[0]
    m = jnp.dot(s, wmsgT_ref[...], preferred_element_type=jnp.float32)
    wihT = wihT_ref[...]
    gates = (jnp.dot(x_ref[...], wihT[:H], preferred_element_type=jnp.float32)
             + jnp.dot(m, wihT[H:], preferred_element_type=jnp.float32)
             + jnp.dot(h_ref[...], whhT_ref[...], preferred_element_type=jnp.float32))
    gi = gates[:, :H]
    gf = gates[:, H:2 * H]
    gg = gates[:, 2 * H:3 * H]
    go = gates[:, 3 * H:]
    c_new = jax.nn.sigmoid(gf) * c_ref[...] + jax.nn.sigmoid(gi) * jnp.tanh(gg)
    h_out[...] = jax.nn.sigmoid(go) * jnp.tanh(c_new)
    c_out[...] = c_new


def _tc_lstm(x, sc_out, h, c, wmsgT, wihT, whhT):
    row_spec = pl.BlockSpec((BLK, H), lambda i: (i, 0))
    full2 = lambda shape: pl.BlockSpec(shape, lambda i: (0, 0))
    return pl.pallas_call(
        _lstm_block,
        grid=(N // BLK,),
        in_specs=[
            row_spec,
            pl.BlockSpec((1, BLK, H), lambda i: (0, i, 0)),
            pl.BlockSpec((1, BLK, H), lambda i: (1, i, 0)),
            row_spec,
            row_spec,
            full2((H, H)),
            full2((2 * H, 4 * H)),
            full2((H, 4 * H)),
        ],
        out_specs=[row_spec, row_spec],
        out_shape=[jax.ShapeDtypeStruct((N, H), jnp.float32),
                   jax.ShapeDtypeStruct((N, H), jnp.float32)],
    )(x, sc_out, sc_out, h, c, wmsgT, wihT, whhT)


def _gol_sum(v, sel):
    n2 = jnp.sum(v * v, axis=1, keepdims=True)
    inv = 1.0 / (jnp.sqrt(n2) + 1e-8)
    vn = v * inv
    bs = jnp.dot(sel, vn, preferred_element_type=jnp.float32)
    tr = jnp.sum(sel * jnp.transpose(n2 * inv * inv))
    return (jnp.sum(bs * bs) - tr) / (BOARD * (BOARD - 1))


def _final_block(h1_ref, h2_ref, h3_ref, emb_ref, woutT_ref,
                 gol_out, in_out, out_out):
    i = pl.program_id(0)
    h3v = h3_ref[...]
    y = jnp.dot(h3v, woutT_ref[...], preferred_element_type=jnp.float32)
    rows = lax.broadcasted_iota(jnp.int32, (BLK, 1), 0)
    mask0 = (rows % (BOARD + 1)) == 0
    embv = emb_ref[...]
    in_out[...] = jnp.where(mask0, embv, h3v)
    out_out[...] = jnp.where(mask0, embv, y)

    colid = lax.broadcasted_iota(jnp.int32, (NB_BLK, BLK), 1)
    rowid = lax.broadcasted_iota(jnp.int32, (NB_BLK, BLK), 0)
    sel = jnp.where((colid // (BOARD + 1) == rowid)
                    & (colid % (BOARD + 1) != 0), 1.0, 0.0)

    part = ((_gol_sum(h1_ref[...], sel) + _gol_sum(h2_ref[...], sel)
             + _gol_sum(h3v, sel)) / (T * B)
            + _gol_sum(y, sel) / B)

    @pl.when(i == 0)
    def _():
        gol_out[...] = jnp.zeros_like(gol_out)

    gol_out[...] += part


def _tc_final(h1, h2, h3, emb_n, woutT):
    row_spec = pl.BlockSpec((BLK, H), lambda i: (i, 0))
    return pl.pallas_call(
        _final_block,
        grid=(N // BLK,),
        in_specs=[row_spec, row_spec, row_spec, row_spec,
                  pl.BlockSpec((H, H), lambda i: (0, 0))],
        out_specs=[pl.BlockSpec((1, 1), lambda i: (0, 0)), row_spec, row_spec],
        out_shape=[jax.ShapeDtypeStruct((1, 1), jnp.float32),
                   jax.ShapeDtypeStruct((N, H), jnp.float32),
                   jax.ShapeDtypeStruct((N, H), jnp.float32)],
    )(h1, h2, h3, emb_n, woutT)


def kernel(node_id, node_sno, edge_index, fixed_embeddings, W_msg, W_ih, W_hh, W_out):
    emb_n = fixed_embeddings[:N]

    lookup_at = (node_id + (BOARD + 1) * node_sno).astype(jnp.int32)
    idx_pad = jnp.pad(lookup_at, (0, LPAD - N)).reshape(LCHUNKS, CHUNK)
    x_pad = _sc_lookup(fixed_embeddings, idx_pad)
    x = x_pad[:N]

    src = edge_index[0].astype(jnp.int32).reshape(NW, E // NW)
    dst = edge_index[1].astype(jnp.int32).reshape(NW, E // NW)
    pad_e = ECH_PER_TILE * CHUNK - E // NW
    src_pad = jnp.pad(src, ((0, 0), (0, pad_e))).reshape(NW, ECH_PER_TILE, CHUNK)
    dst_pad = jnp.pad(dst, ((0, 0), (0, pad_e)),
                      constant_values=N).reshape(NW, ECH_PER_TILE, CHUNK)
    zeros_blk = jnp.zeros((CHUNK, H), jnp.float32)

    wmsgT = W_msg.T
    wihT = W_ih.T
    whhT = W_hh.T

    h = x
    c = x
    steps = []
    for _ in range(T):
        sc_out = _sc_segsum(h, src_pad, dst_pad, zeros_blk)
        h, c = _tc_lstm(x, sc_out, h, c, wmsgT, wihT, whhT)
        steps.append(h)

    gol_arr, in_final, out_final = _tc_final(steps[0], steps[1], steps[2],
                                             emb_n, W_out.T)
    gol = gol_arr[0, 0]
    step_input = jnp.stack(steps, axis=0)
    return (gol, emb_n, in_final, out_final, step_input)
